# baseline (speedup 1.0000x reference)
import functools

import jax
import jax.numpy as jnp
from jax import lax
from jax.experimental import pallas as pl
from jax.experimental.pallas import tpu as pltpu
from jax.experimental.pallas import tpu_sc as plsc

_NC, _NS, _L = 2, 16, 16
_NW = _NC * _NS
_BQ = 256
_KB = 8


def _table_body(q_ref, pe_ref, tab_ref, *, npos):
    t = jnp.dot(q_ref[...], pe_ref[...], preferred_element_type=jnp.float32)
    d = pltpu.roll(t, npos - 1, axis=1) - t
    tab_ref[...] = jnp.concatenate([t, d], axis=1)


def _sc_body(a_hbm, tab_hbm, o_hbm, a_v, o_v, g_v, tab_v, *, rows_pw, skv, npos):
    wid = lax.axis_index("s") * _NC + lax.axis_index("c")
    base = wid * rows_pw
    nslice = skv // _L

    def batch_body(bi, carry0):
        r0 = base + bi * _KB
        pltpu.sync_copy(a_hbm.at[pl.ds(r0, _KB)], a_v)
        pltpu.sync_copy(tab_hbm.at[pl.ds(r0, _KB)], tab_v)

        def row_body(r, carry1):
            def p1(j, tot):
                x = a_v[r, pl.ds(j * _L, _L)]
                g = 1.0 / (1.0 + jnp.exp(-x))
                g_v[pl.ds(j * _L, _L)] = g
                return tot + jnp.sum(g)

            tot = lax.fori_loop(0, nslice, p1, jnp.float32(0.0))

            def p2(j, rem):
                g = g_v[pl.ds(j * _L, _L)]
                pr = plsc.cumsum(g)
                pos = (rem - pr) + g
                pos = jnp.minimum(pos, float(npos - 1))
                idx = pos.astype(jnp.int32)
                fr = pos - idx.astype(jnp.float32)
                tv = plsc.load_gather(tab_v.at[r], [idx])
                dv = plsc.load_gather(tab_v.at[r], [idx + npos])
                o_v[r, pl.ds(j * _L, _L)] = tv + fr * dv
                return rem - jnp.sum(g)

            lax.fori_loop(0, nslice, p2, tot)
            return carry1

        lax.fori_loop(0, _KB, row_body, jnp.int32(0))
        pltpu.sync_copy(o_v, o_hbm.at[pl.ds(r0, _KB)])
        return carry0

    lax.fori_loop(0, rows_pw // _KB, batch_body, jnp.int32(0))


def kernel(query, attn_logits, pos_emb):
    b, h, sq, dim = query.shape
    skv = attn_logits.shape[-1]
    npos = pos_emb.shape[-1]
    rows = b * h * sq
    q2 = query.reshape(rows, dim)
    a2 = attn_logits.reshape(rows, skv)
    pe = pos_emb.reshape(dim, npos)

    tab = pl.pallas_call(
        functools.partial(_table_body, npos=npos),
        grid=(rows // _BQ,),
        in_specs=[
            pl.BlockSpec((_BQ, dim), lambda i: (i, 0)),
            pl.BlockSpec((dim, npos), lambda i: (0, 0)),
        ],
        out_specs=pl.BlockSpec((_BQ, 2 * npos), lambda i: (i, 0)),
        out_shape=jax.ShapeDtypeStruct((rows, 2 * npos), jnp.float32),
    )(q2, pe)

    mesh = plsc.VectorSubcoreMesh(
        core_axis_name="c", subcore_axis_name="s", num_cores=_NC, num_subcores=_NS
    )
    body = functools.partial(
        _sc_body, rows_pw=rows // _NW, skv=skv, npos=npos
    )
    sck = pl.kernel(
        body,
        out_type=jax.ShapeDtypeStruct((rows, skv), jnp.float32),
        mesh=mesh,
        compiler_params=pltpu.CompilerParams(needs_layout_passes=False),
        scratch_types=[
            pltpu.VMEM((_KB, skv), jnp.float32),
            pltpu.VMEM((_KB, skv), jnp.float32),
            pltpu.VMEM((skv,), jnp.float32),
            pltpu.VMEM((_KB, 2 * npos), jnp.float32),
        ],
    )
    out = sck(a2, tab)
    return out.reshape(b, h, sq, skv)


# speedup vs baseline: 18.0049x; 18.0049x over previous
import functools

import jax
import jax.numpy as jnp
import numpy as np
from jax.experimental import pallas as pl
from jax.experimental.pallas import tpu as pltpu

_BQ = 512
_C = 128


def _suffix_weights() -> np.ndarray:
    i = np.arange(_C)
    ut = (i[:, None] >= i[None, :]).astype(np.float32)
    r = np.zeros((2 * _C, 2 * _C), np.float32)
    r[:_C, :_C] = ut
    r[_C:, :_C] = 1.0
    r[_C:, _C:] = ut
    return r


def _cope_body(q_ref, a_ref, pe_ref, w_ref, o_ref, *, skv: int, npos: int):
    t = jnp.dot(q_ref[...], pe_ref[...], preferred_element_type=jnp.float32)
    d = pltpu.roll(t, npos - 1, axis=1) - t
    tw = jax.lax.bitcast_convert_type(
        t.astype(jnp.bfloat16), jnp.uint16
    ).astype(jnp.uint32)
    dw = jax.lax.bitcast_convert_type(
        d.astype(jnp.bfloat16), jnp.uint16
    ).astype(jnp.uint32)
    packed = (tw << 16) | dw
    packed2 = jnp.concatenate([packed, packed], axis=1)

    g = 0.5 * jnp.tanh(a_ref[...] * 0.5) + 0.5
    g_hi = g.astype(jnp.bfloat16)
    g_lo = (g - g_hi.astype(jnp.float32)).astype(jnp.bfloat16)
    w = w_ref[...]

    npair = skv // (2 * _C)
    pairs = []
    for p in range(npair):
        lo, hi = p * 2 * _C, (p + 1) * 2 * _C
        acc = jnp.dot(g_hi[:, lo:hi], w, preferred_element_type=jnp.float32)
        acc = acc + jnp.dot(g_lo[:, lo:hi], w, preferred_element_type=jnp.float32)
        pairs.append(acc)

    carry = jnp.zeros((pairs[0].shape[0], 1), jnp.float32)
    s_list = [None] * npair
    for p in range(npair - 1, -1, -1):
        s_list[p] = pairs[p] + carry
        carry = carry + pairs[p][:, 0:1]
    for p in range(npair):
        pos = jnp.minimum(s_list[p], float(npos - 1))
        idx = pos.astype(jnp.int32)
        frac = pos - idx.astype(jnp.float32)
        w2 = jnp.take_along_axis(packed2, idx, axis=1, mode="promise_in_bounds")
        tv = jax.lax.bitcast_convert_type(w2 & jnp.uint32(0xFFFF0000), jnp.float32)
        dv = jax.lax.bitcast_convert_type(w2 << 16, jnp.float32)
        o_ref[:, p * 2 * _C : (p + 1) * 2 * _C] = tv + frac * dv


def kernel(query, attn_logits, pos_emb):
    b, h, sq, dim = query.shape
    skv = attn_logits.shape[-1]
    npos = pos_emb.shape[-1]
    rows = b * h * sq
    q2 = query.reshape(rows, dim)
    a2 = attn_logits.reshape(rows, skv)
    pe = pos_emb.reshape(dim, npos)
    w = jnp.asarray(_suffix_weights(), dtype=jnp.bfloat16)

    body = functools.partial(_cope_body, skv=skv, npos=npos)
    out = pl.pallas_call(
        body,
        grid=(rows // _BQ,),
        in_specs=[
            pl.BlockSpec((_BQ, dim), lambda i: (i, 0)),
            pl.BlockSpec((_BQ, skv), lambda i: (i, 0)),
            pl.BlockSpec((dim, npos), lambda i: (0, 0)),
            pl.BlockSpec((2 * _C, 2 * _C), lambda i: (0, 0)),
        ],
        out_specs=pl.BlockSpec((_BQ, skv), lambda i: (i, 0)),
        out_shape=jax.ShapeDtypeStruct((rows, skv), jnp.float32),
    )(q2, a2, pe, w)
    return out.reshape(b, h, sq, skv)
